# baseline (speedup 1.0000x reference)
import functools

import jax
import jax.numpy as jnp
from jax import lax
from jax.experimental import pallas as pl
from jax.experimental.pallas import tpu as pltpu
from jax.experimental.pallas import tpu_sc as plsc

_VOCAB = 1000000
_DIM = 32
_B = 16384
_NEG = 20

_info = plsc.get_sparse_core_info()
_NC, _NS, _L = _info.num_cores, _info.num_subcores, _info.num_lanes
_NW = _NC * _NS
_BPW = _B // _NW
_CB = 128
_NCHUNK = _BPW // _CB
_NBLK = _NW * _NCHUNK


def _sc_scores(u_t, v_t, logp, pos_u, pos_v, neg_t):
    mesh = plsc.VectorSubcoreMesh(core_axis_name="c", subcore_axis_name="s")

    @functools.partial(
        pl.kernel,
        mesh=mesh,
        compiler_params=pltpu.CompilerParams(
            needs_layout_passes=False, use_tc_tiling_on_sc=False),
        out_type=[
            jax.ShapeDtypeStruct((_NBLK, _CB), jnp.float32),
            jax.ShapeDtypeStruct((_NBLK, _NEG, _CB), jnp.float32),
        ],
        scratch_types=[
            pltpu.VMEM((_CB,), jnp.int32),
            pltpu.VMEM((_CB,), jnp.int32),
            pltpu.VMEM((_NEG, _CB), jnp.int32),
            pltpu.VMEM((_DIM, _CB), jnp.float32),
            pltpu.VMEM((_DIM, _CB), jnp.float32),
            pltpu.VMEM((_DIM, _NEG, _CB), jnp.float32),
            pltpu.VMEM((_CB,), jnp.float32),
            pltpu.VMEM((_CB,), jnp.float32),
            pltpu.VMEM((_NEG, _CB), jnp.float32),
            pltpu.SemaphoreType.DMA,
        ],
    )
    def sc_k(u_hbm, v_hbm, lp_hbm, pu_hbm, pv_hbm, nt_hbm,
             pos_out, neg_out,
             idx_u, idx_v, idx_n, u_buf, v_buf, n_buf, lp_v,
             pos_st, neg_st, sem):
        wid = lax.axis_index("s") * _NC + lax.axis_index("c")

        def chunk_body(ci, carry):
            base = wid * _BPW + ci * _CB
            blk = wid * _NCHUNK + ci
            pltpu.sync_copy(pu_hbm.at[pl.ds(base, _CB)], idx_u)
            pltpu.sync_copy(pv_hbm.at[pl.ds(base, _CB)], idx_v)
            for n in range(_NEG):
                pltpu.sync_copy(nt_hbm.at[n].at[pl.ds(base, _CB)],
                                idx_n.at[n])
            pltpu.async_copy(lp_hbm.at[idx_u], lp_v, sem)

            def fire(d, c):
                pltpu.async_copy(u_hbm.at[d].at[idx_u], u_buf.at[d], sem)
                pltpu.async_copy(v_hbm.at[d].at[idx_v], v_buf.at[d], sem)
                for n in range(_NEG):
                    pltpu.async_copy(v_hbm.at[d].at[idx_n.at[n]],
                                     n_buf.at[d, n], sem)
                return c

            lax.fori_loop(0, _DIM, fire, 0)

            def drain(d, c):
                pltpu.make_async_copy(
                    u_hbm.at[d].at[idx_u], u_buf.at[d], sem).wait()
                pltpu.make_async_copy(
                    v_hbm.at[d].at[idx_v], v_buf.at[d], sem).wait()
                for n in range(_NEG):
                    pltpu.make_async_copy(
                        v_hbm.at[d].at[idx_n.at[n]], n_buf.at[d, n],
                        sem).wait()
                return c

            lax.fori_loop(0, _DIM, drain, 0)
            pltpu.make_async_copy(lp_hbm.at[idx_u], lp_v, sem).wait()

            def group_body(g, carry2):
                gsl = pl.ds(g * _L, _L)
                us = [u_buf[d, gsl] for d in range(_DIM)]
                quad_h = us[0] * us[0]
                for d in range(1, _DIM):
                    quad_h = quad_h + us[d] * us[d]
                linacc = us[0] * v_buf[0, gsl]
                for d in range(1, _DIM):
                    linacc = linacc + us[d] * v_buf[d, gsl]
                bias = lp_v[gsl] - 0.5 * quad_h
                pos_st[gsl] = linacc + bias

                def n_body(n, carry3):
                    acc = us[0] * n_buf.at[0][n, gsl]
                    for d in range(1, _DIM):
                        acc = acc + us[d] * n_buf.at[d][n, gsl]
                    neg_st[n, gsl] = acc + bias
                    return carry3

                lax.fori_loop(0, _NEG, n_body, 0)
                return carry2

            lax.fori_loop(0, _CB // _L, group_body, 0)
            pltpu.sync_copy(pos_st, pos_out.at[blk])
            pltpu.sync_copy(neg_st, neg_out.at[blk])
            return carry

        lax.fori_loop(0, _NCHUNK, chunk_body, 0)

    return sc_k(u_t, v_t, logp, pos_u, pos_v, neg_t)


def _tc_reduce(pos_raw, neg_raw):
    def body(pos_ref, neg_ref, out_ref):
        p = jnp.clip(pos_ref[...], -10.0, 10.0)
        q = jnp.clip(neg_ref[...], -10.0, 10.0)
        tot = jnp.sum(jnp.log1p(jnp.exp(-p))) + jnp.sum(jnp.log1p(jnp.exp(q)))
        out_ref[0, 0] = tot * (1.0 / _B)

    return pl.pallas_call(
        body,
        out_shape=jax.ShapeDtypeStruct((1, 1), jnp.float32),
        out_specs=pl.BlockSpec(memory_space=pltpu.SMEM),
    )(pos_raw, neg_raw)


def kernel(u_weight, v_weight, log_priors, pos_u, pos_v, neg_v):
    u_t = u_weight.T
    v_t = v_weight.T
    neg_t = neg_v.astype(jnp.int32).T
    pos_raw, neg_raw = _sc_scores(
        u_t, v_t, log_priors,
        pos_u.astype(jnp.int32), pos_v.astype(jnp.int32), neg_t)
    out = _tc_reduce(pos_raw, neg_raw)
    return out.reshape(())


# speedup vs baseline: 7.1039x; 7.1039x over previous
import functools

import jax
import jax.numpy as jnp
from jax import lax
from jax.experimental import pallas as pl
from jax.experimental.pallas import tpu as pltpu
from jax.experimental.pallas import tpu_sc as plsc

_VOCAB = 1000000
_DIM = 32
_B = 16384
_NEG = 20

_info = plsc.get_sparse_core_info()
_NC, _NS, _L = _info.num_cores, _info.num_subcores, _info.num_lanes
_NW = _NC * _NS
_BPW = _B // _NW
_CB = 32
_NCHUNK = _BPW // _CB
_CE = _CB * _NEG

_CIN = 8192
_M = _CIN // 512
_GRID = -(-_VOCAB // _CIN)
_Q = _GRID * (_CIN // 4)


def _compact(t_table):
    def body(in_ref, out_ref):
        for m in range(_M):
            x = in_ref[:, m * 512:(m + 1) * 512]
            y = x.T
            z = jnp.concatenate(
                [y[0:128], y[128:256], y[256:384], y[384:512]], axis=1)
            out_ref[m * 128:(m + 1) * 128, :] = z

    return pl.pallas_call(
        body,
        grid=(_GRID,),
        in_specs=[pl.BlockSpec((_DIM, _CIN), lambda j: (0, j))],
        out_specs=pl.BlockSpec((_CIN // 4, 128), lambda j: (j, 0)),
        out_shape=jax.ShapeDtypeStruct((_Q, 128), jnp.float32),
    )(t_table)


def _sc_scores(u_c, v_c, logp, pos_u, pos_v, neg_flat):
    mesh = plsc.VectorSubcoreMesh(core_axis_name="c", subcore_axis_name="s")

    @functools.partial(
        pl.kernel,
        mesh=mesh,
        compiler_params=pltpu.CompilerParams(
            needs_layout_passes=False, use_tc_tiling_on_sc=False),
        out_type=[
            jax.ShapeDtypeStruct((_B,), jnp.float32),
            jax.ShapeDtypeStruct((_B * _NEG,), jnp.float32),
        ],
        scratch_types=[
            pltpu.VMEM((_CB,), jnp.int32),
            pltpu.VMEM((_CB,), jnp.int32),
            pltpu.VMEM((_CE,), jnp.int32),
            pltpu.VMEM((_CB,), jnp.int32),
            pltpu.VMEM((_CB,), jnp.int32),
            pltpu.VMEM((_CE,), jnp.int32),
            pltpu.VMEM((_CB,), jnp.int32),
            pltpu.VMEM((_CB,), jnp.int32),
            pltpu.VMEM((_CE,), jnp.int32),
            pltpu.VMEM((_CB, 128), jnp.float32),
            pltpu.VMEM((_CB, 128), jnp.float32),
            pltpu.VMEM((_CE, 128), jnp.float32),
            pltpu.VMEM((_CB,), jnp.float32),
            pltpu.VMEM((_CB,), jnp.float32),
            pltpu.VMEM((_CE,), jnp.float32),
            pltpu.SemaphoreType.DMA,
        ],
    )
    def sc_k(u_hbm, v_hbm, lp_hbm, pu_hbm, pv_hbm, nf_hbm,
             pos_out, neg_out,
             ridx_u, ridx_v, ridx_n, q_u, q_v, q_n, o_u, o_v, o_n,
             u_rows, v_rows, n_rows, lp_v, pos_st, neg_st, sem):
        wid = lax.axis_index("s") * _NC + lax.axis_index("c")
        lane = lax.iota(jnp.int32, _L)

        def pack_ids(raw_ref, q_ref, o_ref, count):
            def grp(i, c):
                sl = pl.ds(i * _L, _L)
                r = raw_ref[sl]
                q_ref[sl] = ((r >> 9) << 7) | (r & 127)
                o_ref[sl] = ((r >> 7) & 3) << 5
                return c
            lax.fori_loop(0, count // _L, grp, 0)

        def chunk_body(ci, carry):
            base = wid * _BPW + ci * _CB
            pltpu.sync_copy(pu_hbm.at[pl.ds(base, _CB)], ridx_u)
            pltpu.sync_copy(pv_hbm.at[pl.ds(base, _CB)], ridx_v)
            pltpu.sync_copy(nf_hbm.at[pl.ds(base * _NEG, _CE)], ridx_n)
            pack_ids(ridx_u, q_u, o_u, _CB)
            pack_ids(ridx_v, q_v, o_v, _CB)
            pack_ids(ridx_n, q_n, o_n, _CE)
            cps = [
                pltpu.async_copy(lp_hbm.at[ridx_u], lp_v, sem),
                pltpu.async_copy(u_hbm.at[q_u], u_rows, sem),
                pltpu.async_copy(v_hbm.at[q_v], v_rows, sem),
            ]
            for j in range(_CE // 128):
                cps.append(pltpu.async_copy(
                    v_hbm.at[q_n.at[pl.ds(j * 128, 128)]],
                    n_rows.at[pl.ds(j * 128, 128)], sem))
            for cp in cps:
                cp.wait()

            def group_body(g, carry2):
                gsl = pl.ds(g * _L, _L)
                bvec = g * _L + lane
                su = o_u[gsl]
                sv = o_v[gsl]
                us = [plsc.load_gather(u_rows, [bvec, su + d])
                      for d in range(_DIM)]
                quad_h = us[0] * us[0]
                for d in range(1, _DIM):
                    quad_h = quad_h + us[d] * us[d]
                v0 = plsc.load_gather(v_rows, [bvec, sv])
                linacc = us[0] * v0
                for d in range(1, _DIM):
                    vv = plsc.load_gather(v_rows, [bvec, sv + d])
                    linacc = linacc + us[d] * vv
                bias = lp_v[gsl] - 0.5 * quad_h
                pos_st[gsl] = linacc + bias

                def n_body(n, carry3):
                    evec = bvec * _NEG + n
                    sn = plsc.load_gather(o_n, [evec])
                    acc = us[0] * plsc.load_gather(n_rows, [evec, sn])
                    for d in range(1, _DIM):
                        acc = acc + us[d] * plsc.load_gather(
                            n_rows, [evec, sn + d])
                    plsc.store_scatter(neg_st, [evec], acc + bias)
                    return carry3

                lax.fori_loop(0, _NEG, n_body, 0)
                return carry2

            lax.fori_loop(0, _CB // _L, group_body, 0)
            pltpu.sync_copy(pos_st, pos_out.at[pl.ds(base, _CB)])
            pltpu.sync_copy(neg_st, neg_out.at[pl.ds(base * _NEG, _CE)])
            return carry

        lax.fori_loop(0, _NCHUNK, chunk_body, 0)

    return sc_k(u_c, v_c, logp, pos_u, pos_v, neg_flat)


def _tc_reduce(pos_raw, neg_raw):
    def body(pos_ref, neg_ref, out_ref):
        p = jnp.clip(pos_ref[...], -10.0, 10.0)
        q = jnp.clip(neg_ref[...], -10.0, 10.0)
        tot = jnp.sum(jnp.log1p(jnp.exp(-p))) + jnp.sum(jnp.log1p(jnp.exp(q)))
        out_ref[0, 0] = tot * (1.0 / _B)

    return pl.pallas_call(
        body,
        out_shape=jax.ShapeDtypeStruct((1, 1), jnp.float32),
        out_specs=pl.BlockSpec(memory_space=pltpu.SMEM),
    )(pos_raw, neg_raw)


def kernel(u_weight, v_weight, log_priors, pos_u, pos_v, neg_v):
    u_c = _compact(u_weight.T)
    v_c = _compact(v_weight.T)
    neg_flat = neg_v.reshape(-1).astype(jnp.int32)
    pos_raw, neg_raw = _sc_scores(
        u_c, v_c, log_priors,
        pos_u.astype(jnp.int32), pos_v.astype(jnp.int32), neg_flat)
    out = _tc_reduce(pos_raw.reshape(_B // 128, 128),
                     neg_raw.reshape(_B * _NEG // 128, 128))
    return out.reshape(())


# speedup vs baseline: 7.4239x; 1.0451x over previous
import functools

import jax
import jax.numpy as jnp
from jax import lax
from jax.experimental import pallas as pl
from jax.experimental.pallas import tpu as pltpu
from jax.experimental.pallas import tpu_sc as plsc

_VOCAB = 1000000
_DIM = 32
_B = 16384
_NEG = 20

_info = plsc.get_sparse_core_info()
_NC, _NS, _L = _info.num_cores, _info.num_subcores, _info.num_lanes
_NW = _NC * _NS
_BPW = _B // _NW
_CB = 32
_NCHUNK = _BPW // _CB
_CE = _CB * _NEG

_CIN = 8192
_M = _CIN // 512
_GRID = -(-_VOCAB // _CIN)
_Q = _GRID * (_CIN // 4)


def _compact2(ut, vt):
    def one(in_ref, out_ref):
        for m in range(_M):
            x = in_ref[:, m * 512:(m + 1) * 512]
            y = x.T
            z = jnp.concatenate(
                [y[0:128], y[128:256], y[256:384], y[384:512]], axis=1)
            out_ref[m * 128:(m + 1) * 128, :] = z

    def body(u_ref, v_ref, uo_ref, vo_ref):
        one(u_ref, uo_ref)
        one(v_ref, vo_ref)

    return pl.pallas_call(
        body,
        grid=(_GRID,),
        in_specs=[pl.BlockSpec((_DIM, _CIN), lambda j: (0, j)),
                  pl.BlockSpec((_DIM, _CIN), lambda j: (0, j))],
        out_specs=[pl.BlockSpec((_CIN // 4, 128), lambda j: (j, 0)),
                   pl.BlockSpec((_CIN // 4, 128), lambda j: (j, 0))],
        out_shape=[jax.ShapeDtypeStruct((_Q, 128), jnp.float32),
                   jax.ShapeDtypeStruct((_Q, 128), jnp.float32)],
    )(ut, vt)


def _sc_scores(u_c, v_c, logp, pos_u, pos_v, neg_flat):
    mesh = plsc.VectorSubcoreMesh(core_axis_name="c", subcore_axis_name="s")

    @functools.partial(
        pl.kernel,
        mesh=mesh,
        compiler_params=pltpu.CompilerParams(
            needs_layout_passes=False, use_tc_tiling_on_sc=False),
        out_type=[
            jax.ShapeDtypeStruct((_B,), jnp.float32),
            jax.ShapeDtypeStruct((_B * _NEG,), jnp.float32),
        ],
        scratch_types=[
            pltpu.VMEM((_BPW,), jnp.int32),
            pltpu.VMEM((_BPW,), jnp.int32),
            pltpu.VMEM((_BPW * _NEG,), jnp.int32),
            pltpu.VMEM((_BPW,), jnp.int32),
            pltpu.VMEM((_BPW,), jnp.int32),
            pltpu.VMEM((_BPW * _NEG,), jnp.int32),
            pltpu.VMEM((_BPW,), jnp.int32),
            pltpu.VMEM((_BPW,), jnp.int32),
            pltpu.VMEM((_BPW * _NEG,), jnp.int32),
            pltpu.VMEM((_CB, 128), jnp.float32),
            pltpu.VMEM((_CB, 128), jnp.float32),
            pltpu.VMEM((_CE, 128), jnp.float32),
            pltpu.VMEM((_BPW,), jnp.float32),
            pltpu.VMEM((_CB,), jnp.float32),
            pltpu.VMEM((_CE,), jnp.float32),
            pltpu.SemaphoreType.DMA,
        ],
    )
    def sc_k(u_hbm, v_hbm, lp_hbm, pu_hbm, pv_hbm, nf_hbm,
             pos_out, neg_out,
             ridx_u, ridx_v, ridx_n, q_u, q_v, q_n, o_u, o_v, o_n,
             u_rows, v_rows, n_rows, lp_v, pos_st, neg_st, sem):
        wid = lax.axis_index("s") * _NC + lax.axis_index("c")
        lane = lax.iota(jnp.int32, _L)
        wbase = wid * _BPW

        pltpu.sync_copy(pu_hbm.at[pl.ds(wbase, _BPW)], ridx_u)
        pltpu.sync_copy(pv_hbm.at[pl.ds(wbase, _BPW)], ridx_v)
        pltpu.sync_copy(nf_hbm.at[pl.ds(wbase * _NEG, _BPW * _NEG)], ridx_n)
        cp_lp = pltpu.async_copy(lp_hbm.at[ridx_u], lp_v, sem)

        def pack_ids(raw_ref, q_ref, o_ref, count):
            def grp(i, c):
                sl = pl.ds(i * _L, _L)
                r = raw_ref[sl]
                q_ref[sl] = ((r >> 9) << 7) | (r & 127)
                o_ref[sl] = ((r >> 7) & 3) << 5
                return c
            lax.fori_loop(0, count // _L, grp, 0)

        pack_ids(ridx_u, q_u, o_u, _BPW)
        pack_ids(ridx_v, q_v, o_v, _BPW)
        pack_ids(ridx_n, q_n, o_n, _BPW * _NEG)
        cp_lp.wait()

        def chunk_body(ci, carry):
            base = wbase + ci * _CB
            cb0 = ci * _CB
            cps = [
                pltpu.async_copy(u_hbm.at[q_u.at[pl.ds(cb0, _CB)]],
                                 u_rows, sem),
                pltpu.async_copy(v_hbm.at[q_v.at[pl.ds(cb0, _CB)]],
                                 v_rows, sem),
            ]
            for j in range(_CE // 128):
                cps.append(pltpu.async_copy(
                    v_hbm.at[q_n.at[pl.ds(cb0 * _NEG + j * 128, 128)]],
                    n_rows.at[pl.ds(j * 128, 128)], sem))
            for cp in cps:
                cp.wait()

            def group_body(g, carry2):
                gsl = pl.ds(cb0 + g * _L, _L)
                bvec = g * _L + lane
                su = o_u[gsl]
                sv = o_v[gsl]
                us = [plsc.load_gather(u_rows, [bvec, su + d])
                      for d in range(_DIM)]
                quad_h = us[0] * us[0]
                for d in range(1, _DIM):
                    quad_h = quad_h + us[d] * us[d]
                v0 = plsc.load_gather(v_rows, [bvec, sv])
                linacc = us[0] * v0
                for d in range(1, _DIM):
                    vv = plsc.load_gather(v_rows, [bvec, sv + d])
                    linacc = linacc + us[d] * vv
                bias = lp_v[gsl] - 0.5 * quad_h
                pos_st[pl.ds(g * _L, _L)] = linacc + bias

                def n_body(n, carry3):
                    evec = bvec * _NEG + n
                    sn = plsc.load_gather(o_n, [cb0 * _NEG + evec])
                    acc = us[0] * plsc.load_gather(n_rows, [evec, sn])
                    for d in range(1, _DIM):
                        acc = acc + us[d] * plsc.load_gather(
                            n_rows, [evec, sn + d])
                    plsc.store_scatter(neg_st, [evec], acc + bias)
                    return carry3

                lax.fori_loop(0, _NEG, n_body, 0)
                return carry2

            lax.fori_loop(0, _CB // _L, group_body, 0)
            pltpu.sync_copy(pos_st, pos_out.at[pl.ds(base, _CB)])
            pltpu.sync_copy(neg_st, neg_out.at[pl.ds(base * _NEG, _CE)])
            return carry

        lax.fori_loop(0, _NCHUNK, chunk_body, 0)

    return sc_k(u_c, v_c, logp, pos_u, pos_v, neg_flat)


def _tc_reduce(pos_raw, neg_raw):
    def body(pos_ref, neg_ref, out_ref):
        p = jnp.clip(pos_ref[...], -10.0, 10.0)
        q = jnp.clip(neg_ref[...], -10.0, 10.0)
        tot = jnp.sum(jnp.log1p(jnp.exp(-p))) + jnp.sum(jnp.log1p(jnp.exp(q)))
        out_ref[0, 0] = tot * (1.0 / _B)

    return pl.pallas_call(
        body,
        out_shape=jax.ShapeDtypeStruct((1, 1), jnp.float32),
        out_specs=pl.BlockSpec(memory_space=pltpu.SMEM),
    )(pos_raw, neg_raw)


def kernel(u_weight, v_weight, log_priors, pos_u, pos_v, neg_v):
    u_c, v_c = _compact2(u_weight.T, v_weight.T)
    neg_flat = neg_v.reshape(-1).astype(jnp.int32)
    pos_raw, neg_raw = _sc_scores(
        u_c, v_c, log_priors,
        pos_u.astype(jnp.int32), pos_v.astype(jnp.int32), neg_flat)
    out = _tc_reduce(pos_raw.reshape(_B // 128, 128),
                     neg_raw.reshape(_B * _NEG // 128, 128))
    return out.reshape(())


# speedup vs baseline: 8.1751x; 1.1012x over previous
import functools

import jax
import jax.numpy as jnp
from jax import lax
from jax.experimental import pallas as pl
from jax.experimental.pallas import tpu as pltpu
from jax.experimental.pallas import tpu_sc as plsc

_VOCAB = 1000000
_DIM = 32
_B = 16384
_NEG = 20

_info = plsc.get_sparse_core_info()
_NC, _NS, _L = _info.num_cores, _info.num_subcores, _info.num_lanes
_NW = _NC * _NS
_BPW = _B // _NW
_CB = 16
_NCHUNK = _BPW // _CB
_NPAIR = _NCHUNK // 2
_CE = _CB * _NEG
_NSPLIT = [(0, 128), (128, 128), (256, 64)]

_CIN = 8192
_M = _CIN // 512
_GRID = -(-_VOCAB // _CIN)
_Q = _GRID * (_CIN // 4)


def _compact2(ut, vt):
    def one(in_ref, out_ref):
        for m in range(_M):
            x = in_ref[:, m * 512:(m + 1) * 512]
            y = x.T
            z = jnp.concatenate(
                [y[0:128], y[128:256], y[256:384], y[384:512]], axis=1)
            out_ref[m * 128:(m + 1) * 128, :] = z

    def body(u_ref, v_ref, uo_ref, vo_ref):
        one(u_ref, uo_ref)
        one(v_ref, vo_ref)

    return pl.pallas_call(
        body,
        grid=(_GRID,),
        in_specs=[pl.BlockSpec((_DIM, _CIN), lambda j: (0, j)),
                  pl.BlockSpec((_DIM, _CIN), lambda j: (0, j))],
        out_specs=[pl.BlockSpec((_CIN // 4, 128), lambda j: (j, 0)),
                   pl.BlockSpec((_CIN // 4, 128), lambda j: (j, 0))],
        out_shape=[jax.ShapeDtypeStruct((_Q, 128), jnp.float32),
                   jax.ShapeDtypeStruct((_Q, 128), jnp.float32)],
    )(ut, vt)


def _sc_scores(u_c, v_c, logp, pos_u, pos_v, neg_flat):
    mesh = plsc.VectorSubcoreMesh(core_axis_name="c", subcore_axis_name="s")

    @functools.partial(
        pl.kernel,
        mesh=mesh,
        compiler_params=pltpu.CompilerParams(
            needs_layout_passes=False, use_tc_tiling_on_sc=False),
        out_type=[
            jax.ShapeDtypeStruct((_B,), jnp.float32),
            jax.ShapeDtypeStruct((_B * _NEG,), jnp.float32),
        ],
        scratch_types=[
            pltpu.VMEM((_BPW,), jnp.int32),
            pltpu.VMEM((_BPW,), jnp.int32),
            pltpu.VMEM((_BPW * _NEG,), jnp.int32),
            pltpu.VMEM((_BPW,), jnp.int32),
            pltpu.VMEM((_BPW,), jnp.int32),
            pltpu.VMEM((_BPW * _NEG,), jnp.int32),
            pltpu.VMEM((_BPW,), jnp.int32),
            pltpu.VMEM((_BPW,), jnp.int32),
            pltpu.VMEM((_BPW * _NEG,), jnp.int32),
            pltpu.VMEM((2, _CB, 128), jnp.float32),
            pltpu.VMEM((2, _CB, 128), jnp.float32),
            pltpu.VMEM((2, _CE, 128), jnp.float32),
            pltpu.VMEM((_BPW,), jnp.float32),
            pltpu.VMEM((2, _CB), jnp.float32),
            pltpu.VMEM((2, _CE), jnp.float32),
            pltpu.SemaphoreType.DMA,
            pltpu.SemaphoreType.DMA,
            pltpu.SemaphoreType.DMA,
            pltpu.SemaphoreType.DMA,
        ],
    )
    def sc_k(u_hbm, v_hbm, lp_hbm, pu_hbm, pv_hbm, nf_hbm,
             pos_out, neg_out,
             ridx_u, ridx_v, ridx_n, q_u, q_v, q_n, o_u, o_v, o_n,
             u_rows, v_rows, n_rows, lp_v, pos_st, neg_st,
             gsem0, gsem1, osem0, osem1):
        wid = lax.axis_index("s") * _NC + lax.axis_index("c")
        lane = lax.iota(jnp.int32, _L)
        wbase = wid * _BPW

        pltpu.sync_copy(pu_hbm.at[pl.ds(wbase, _BPW)], ridx_u)
        pltpu.sync_copy(pv_hbm.at[pl.ds(wbase, _BPW)], ridx_v)
        pltpu.sync_copy(nf_hbm.at[pl.ds(wbase * _NEG, _BPW * _NEG)], ridx_n)
        cp_lp = pltpu.async_copy(lp_hbm.at[ridx_u], lp_v, gsem0)

        def pack_ids(raw_ref, q_ref, o_ref, count):
            def grp(i, c):
                sl = pl.ds(i * _L, _L)
                r = raw_ref[sl]
                q_ref[sl] = ((r >> 9) << 7) | (r & 127)
                o_ref[sl] = ((r >> 7) & 3) << 5
                return c
            lax.fori_loop(0, count // _L, grp, 0)

        pack_ids(ridx_u, q_u, o_u, _BPW)
        pack_ids(ridx_v, q_v, o_v, _BPW)
        pack_ids(ridx_n, q_n, o_n, _BPW * _NEG)
        cp_lp.wait()

        def gather_copies(ci, slot, gsem, make_only):
            mk = pltpu.make_async_copy if make_only else pltpu.async_copy
            cb0 = ci * _CB
            cps = [
                mk(u_hbm.at[q_u.at[pl.ds(cb0, _CB)]], u_rows.at[slot], gsem),
                mk(v_hbm.at[q_v.at[pl.ds(cb0, _CB)]], v_rows.at[slot], gsem),
            ]
            for j0, jn in _NSPLIT:
                cps.append(mk(
                    v_hbm.at[q_n.at[pl.ds(cb0 * _NEG + j0, jn)]],
                    n_rows.at[slot].at[pl.ds(j0, jn)], gsem))
            return cps

        def fire(ci, slot, gsem):
            gather_copies(ci, slot, gsem, make_only=False)

        def drain(ci, slot, gsem):
            for cp in gather_copies(ci, slot, gsem, make_only=True):
                cp.wait()

        def wait_out(slot, osem):
            pltpu.make_async_copy(
                pos_st.at[slot], pos_out.at[pl.ds(0, _CB)], osem).wait()
            pltpu.make_async_copy(
                neg_st.at[slot], neg_out.at[pl.ds(0, _CE)], osem).wait()

        def compute(ci, slot, osem, first):
            base = wbase + ci * _CB
            cb0 = ci * _CB
            gsl = pl.ds(cb0, _L)
            bvec = lane
            su = o_u[gsl]
            sv = o_v[gsl]
            ur = u_rows.at[slot]
            vr = v_rows.at[slot]
            nr = n_rows.at[slot]
            us = [plsc.load_gather(ur, [bvec, su + d]) for d in range(_DIM)]
            quad_h = us[0] * us[0]
            for d in range(1, _DIM):
                quad_h = quad_h + us[d] * us[d]
            linacc = us[0] * plsc.load_gather(vr, [bvec, sv])
            for d in range(1, _DIM):
                linacc = linacc + us[d] * plsc.load_gather(vr, [bvec, sv + d])
            bias = lp_v[gsl] - 0.5 * quad_h

            @pl.when(jnp.logical_not(first))
            def _():
                wait_out(slot, osem)

            pos_st[slot] = linacc + bias

            def n_body(n, carry3):
                evec = bvec * _NEG + n
                sn = plsc.load_gather(o_n, [cb0 * _NEG + evec])
                acc = us[0] * plsc.load_gather(nr, [evec, sn])
                for d in range(1, _DIM):
                    acc = acc + us[d] * plsc.load_gather(nr, [evec, sn + d])
                plsc.store_scatter(neg_st.at[slot], [evec], acc + bias)
                return carry3

            lax.fori_loop(0, _NEG, n_body, 0)
            pltpu.async_copy(pos_st.at[slot], pos_out.at[pl.ds(base, _CB)],
                             osem)
            pltpu.async_copy(neg_st.at[slot],
                             neg_out.at[pl.ds(base * _NEG, _CE)], osem)

        fire(0, 0, gsem0)

        def pair_body(cp, carry):
            ci0 = 2 * cp
            fire(ci0 + 1, 1, gsem1)
            drain(ci0, 0, gsem0)
            compute(ci0, 0, osem0, cp == 0)

            @pl.when(cp < _NPAIR - 1)
            def _():
                fire(ci0 + 2, 0, gsem0)

            drain(ci0 + 1, 1, gsem1)
            compute(ci0 + 1, 1, osem1, cp == 0)
            return carry

        lax.fori_loop(0, _NPAIR, pair_body, 0)
        wait_out(0, osem0)
        wait_out(1, osem1)

    return sc_k(u_c, v_c, logp, pos_u, pos_v, neg_flat)


def _tc_reduce(pos_raw, neg_raw):
    def body(pos_ref, neg_ref, out_ref):
        p = jnp.clip(pos_ref[...], -10.0, 10.0)
        q = jnp.clip(neg_ref[...], -10.0, 10.0)
        tot = jnp.sum(jnp.log1p(jnp.exp(-p))) + jnp.sum(jnp.log1p(jnp.exp(q)))
        out_ref[0, 0] = tot * (1.0 / _B)

    return pl.pallas_call(
        body,
        out_shape=jax.ShapeDtypeStruct((1, 1), jnp.float32),
        out_specs=pl.BlockSpec(memory_space=pltpu.SMEM),
    )(pos_raw, neg_raw)


def kernel(u_weight, v_weight, log_priors, pos_u, pos_v, neg_v):
    u_c, v_c = _compact2(u_weight.T, v_weight.T)
    neg_flat = neg_v.reshape(-1).astype(jnp.int32)
    pos_raw, neg_raw = _sc_scores(
        u_c, v_c, log_priors,
        pos_u.astype(jnp.int32), pos_v.astype(jnp.int32), neg_flat)
    out = _tc_reduce(pos_raw.reshape(_B // 128, 128),
                     neg_raw.reshape(_B * _NEG // 128, 128))
    return out.reshape(())


# speedup vs baseline: 8.1946x; 1.0024x over previous
import functools

import jax
import jax.numpy as jnp
from jax import lax
from jax.experimental import pallas as pl
from jax.experimental.pallas import tpu as pltpu
from jax.experimental.pallas import tpu_sc as plsc

_VOCAB = 1000000
_DIM = 32
_B = 16384
_NEG = 20

_info = plsc.get_sparse_core_info()
_NC, _NS, _L = _info.num_cores, _info.num_subcores, _info.num_lanes
_NW = _NC * _NS
_BPW = _B // _NW
_CB = 16
_NCHUNK = _BPW // _CB
_NPAIR = _NCHUNK // 2
_CE = _CB * _NEG
_NSPLIT = [(0, 128), (128, 128), (256, 64)]

_CIN = 16384
_M = _CIN // 512
_GRID = -(-_VOCAB // _CIN)
_Q = _GRID * (_CIN // 4)


def _compact2(ut, vt):
    def one(in_ref, out_ref):
        for m in range(_M):
            x = in_ref[:, m * 512:(m + 1) * 512]
            y = x.T
            z = jnp.concatenate(
                [y[0:128], y[128:256], y[256:384], y[384:512]], axis=1)
            out_ref[m * 128:(m + 1) * 128, :] = z

    def body(u_ref, v_ref, uo_ref, vo_ref):
        one(u_ref, uo_ref)
        one(v_ref, vo_ref)

    return pl.pallas_call(
        body,
        grid=(_GRID,),
        in_specs=[pl.BlockSpec((_DIM, _CIN), lambda j: (0, j)),
                  pl.BlockSpec((_DIM, _CIN), lambda j: (0, j))],
        out_specs=[pl.BlockSpec((_CIN // 4, 128), lambda j: (j, 0)),
                   pl.BlockSpec((_CIN // 4, 128), lambda j: (j, 0))],
        out_shape=[jax.ShapeDtypeStruct((_Q, 128), jnp.float32),
                   jax.ShapeDtypeStruct((_Q, 128), jnp.float32)],
    )(ut, vt)


def _sc_scores(u_c, v_c, logp, pos_u, pos_v, neg_flat):
    mesh = plsc.VectorSubcoreMesh(core_axis_name="c", subcore_axis_name="s")

    @functools.partial(
        pl.kernel,
        mesh=mesh,
        compiler_params=pltpu.CompilerParams(
            needs_layout_passes=False, use_tc_tiling_on_sc=False),
        out_type=[
            jax.ShapeDtypeStruct((_B,), jnp.float32),
            jax.ShapeDtypeStruct((_B * _NEG,), jnp.float32),
        ],
        scratch_types=[
            pltpu.VMEM((_BPW,), jnp.int32),
            pltpu.VMEM((_BPW,), jnp.int32),
            pltpu.VMEM((_BPW * _NEG,), jnp.int32),
            pltpu.VMEM((_BPW,), jnp.int32),
            pltpu.VMEM((_BPW,), jnp.int32),
            pltpu.VMEM((_BPW * _NEG,), jnp.int32),
            pltpu.VMEM((_BPW,), jnp.int32),
            pltpu.VMEM((_BPW,), jnp.int32),
            pltpu.VMEM((_BPW * _NEG,), jnp.int32),
            pltpu.VMEM((2, _CB, 128), jnp.float32),
            pltpu.VMEM((2, _CB, 128), jnp.float32),
            pltpu.VMEM((2, _CE, 128), jnp.float32),
            pltpu.VMEM((_BPW,), jnp.float32),
            pltpu.VMEM((2, _CB), jnp.float32),
            pltpu.VMEM((2, _CE), jnp.float32),
            pltpu.SemaphoreType.DMA,
            pltpu.SemaphoreType.DMA,
            pltpu.SemaphoreType.DMA,
            pltpu.SemaphoreType.DMA,
        ],
    )
    def sc_k(u_hbm, v_hbm, lp_hbm, pu_hbm, pv_hbm, nf_hbm,
             pos_out, neg_out,
             ridx_u, ridx_v, ridx_n, q_u, q_v, q_n, o_u, o_v, o_n,
             u_rows, v_rows, n_rows, lp_v, pos_st, neg_st,
             gsem0, gsem1, osem0, osem1):
        wid = lax.axis_index("s") * _NC + lax.axis_index("c")
        lane = lax.iota(jnp.int32, _L)
        wbase = wid * _BPW

        pltpu.sync_copy(pu_hbm.at[pl.ds(wbase, _BPW)], ridx_u)
        pltpu.sync_copy(pv_hbm.at[pl.ds(wbase, _BPW)], ridx_v)
        pltpu.sync_copy(nf_hbm.at[pl.ds(wbase * _NEG, _BPW * _NEG)], ridx_n)
        cp_lp = pltpu.async_copy(lp_hbm.at[ridx_u], lp_v, gsem0)

        def pack_ids(raw_ref, q_ref, o_ref, count):
            def grp(i, c):
                sl = pl.ds(i * _L, _L)
                r = raw_ref[sl]
                q_ref[sl] = ((r >> 9) << 7) | (r & 127)
                o_ref[sl] = ((r >> 7) & 3) << 5
                return c
            lax.fori_loop(0, count // _L, grp, 0)

        pack_ids(ridx_u, q_u, o_u, _BPW)
        pack_ids(ridx_v, q_v, o_v, _BPW)
        pack_ids(ridx_n, q_n, o_n, _BPW * _NEG)
        cp_lp.wait()

        def gather_copies(ci, slot, gsem, make_only):
            mk = pltpu.make_async_copy if make_only else pltpu.async_copy
            cb0 = ci * _CB
            cps = [
                mk(u_hbm.at[q_u.at[pl.ds(cb0, _CB)]], u_rows.at[slot], gsem),
                mk(v_hbm.at[q_v.at[pl.ds(cb0, _CB)]], v_rows.at[slot], gsem),
            ]
            for j0, jn in _NSPLIT:
                cps.append(mk(
                    v_hbm.at[q_n.at[pl.ds(cb0 * _NEG + j0, jn)]],
                    n_rows.at[slot].at[pl.ds(j0, jn)], gsem))
            return cps

        def fire(ci, slot, gsem):
            gather_copies(ci, slot, gsem, make_only=False)

        def drain(ci, slot, gsem):
            for cp in gather_copies(ci, slot, gsem, make_only=True):
                cp.wait()

        def wait_out(slot, osem):
            pltpu.make_async_copy(
                pos_st.at[slot], pos_out.at[pl.ds(0, _CB)], osem).wait()
            pltpu.make_async_copy(
                neg_st.at[slot], neg_out.at[pl.ds(0, _CE)], osem).wait()

        def compute(ci, slot, osem, first):
            base = wbase + ci * _CB
            cb0 = ci * _CB
            gsl = pl.ds(cb0, _L)
            bvec = lane
            su = o_u[gsl]
            sv = o_v[gsl]
            ur = u_rows.at[slot]
            vr = v_rows.at[slot]
            nr = n_rows.at[slot]
            us = [plsc.load_gather(ur, [bvec, su + d]) for d in range(_DIM)]
            quad_h = us[0] * us[0]
            for d in range(1, _DIM):
                quad_h = quad_h + us[d] * us[d]
            linacc = us[0] * plsc.load_gather(vr, [bvec, sv])
            for d in range(1, _DIM):
                linacc = linacc + us[d] * plsc.load_gather(vr, [bvec, sv + d])
            bias = lp_v[gsl] - 0.5 * quad_h

            @pl.when(jnp.logical_not(first))
            def _():
                wait_out(slot, osem)

            pos_st[slot] = linacc + bias

            def n_body(n, carry3):
                evec = bvec * _NEG + n
                sn = plsc.load_gather(o_n, [cb0 * _NEG + evec])
                acc = us[0] * plsc.load_gather(nr, [evec, sn])
                for d in range(1, _DIM):
                    acc = acc + us[d] * plsc.load_gather(nr, [evec, sn + d])
                plsc.store_scatter(neg_st.at[slot], [evec], acc + bias)
                return carry3

            lax.fori_loop(0, _NEG, n_body, 0)
            pltpu.async_copy(pos_st.at[slot], pos_out.at[pl.ds(base, _CB)],
                             osem)
            pltpu.async_copy(neg_st.at[slot],
                             neg_out.at[pl.ds(base * _NEG, _CE)], osem)

        fire(0, 0, gsem0)

        def pair_body(cp, carry):
            ci0 = 2 * cp
            fire(ci0 + 1, 1, gsem1)
            drain(ci0, 0, gsem0)
            compute(ci0, 0, osem0, cp == 0)

            @pl.when(cp < _NPAIR - 1)
            def _():
                fire(ci0 + 2, 0, gsem0)

            drain(ci0 + 1, 1, gsem1)
            compute(ci0 + 1, 1, osem1, cp == 0)
            return carry

        lax.fori_loop(0, _NPAIR, pair_body, 0)
        wait_out(0, osem0)
        wait_out(1, osem1)

    return sc_k(u_c, v_c, logp, pos_u, pos_v, neg_flat)


def _tc_reduce(pos_raw, neg_raw):
    def body(pos_ref, neg_ref, out_ref):
        p = jnp.clip(pos_ref[...], -10.0, 10.0)
        q = jnp.clip(neg_ref[...], -10.0, 10.0)
        tot = jnp.sum(jnp.log1p(jnp.exp(-p))) + jnp.sum(jnp.log1p(jnp.exp(q)))
        out_ref[0, 0] = tot * (1.0 / _B)

    return pl.pallas_call(
        body,
        out_shape=jax.ShapeDtypeStruct((1, 1), jnp.float32),
        out_specs=pl.BlockSpec(memory_space=pltpu.SMEM),
    )(pos_raw, neg_raw)


def kernel(u_weight, v_weight, log_priors, pos_u, pos_v, neg_v):
    u_c, v_c = _compact2(u_weight.T, v_weight.T)
    neg_flat = neg_v.reshape(-1).astype(jnp.int32)
    pos_raw, neg_raw = _sc_scores(
        u_c, v_c, log_priors,
        pos_u.astype(jnp.int32), pos_v.astype(jnp.int32), neg_flat)
    out = _tc_reduce(pos_raw.reshape(_B // 128, 128),
                     neg_raw.reshape(_B * _NEG // 128, 128))
    return out.reshape(())
